# Initial kernel scaffold; baseline (speedup 1.0000x reference)
import jax
import jax.numpy as jnp
from jax.experimental import pallas as pl


def kernel(x, segment_ids, stage_features, W_af1, b_af1, W_af2, b_af2, W_c1, b_c1, W_c2, b_c2):
    raise NotImplementedError("write your pallas kernel here")


# speedup vs baseline: 1.5476x; 1.5476x over previous
import functools

import jax
import jax.numpy as jnp
from jax import lax
from jax.experimental import pallas as pl
from jax.experimental.pallas import tpu as pltpu
from jax.experimental.pallas import tpu_sc as plsc

G = 10000
NC = 2
NS = 16
NW = NC * NS
GPW = 312
GLAST = G - (NW - 1) * GPW
CH = 80
CNT_W = 16


def _mlp1_body(x_ref, sf_ref, w1x_ref, w1s_ref, b1_ref, w2_ref, b2_ref, out_ref):
    h = (jnp.dot(x_ref[...], w1x_ref[...], preferred_element_type=jnp.float32)
         + jnp.dot(sf_ref[...], w1s_ref[...], preferred_element_type=jnp.float32)
         + b1_ref[...])
    g = jax.nn.gelu(h)
    out_ref[...] = (jnp.dot(g, w2_ref[...], preferred_element_type=jnp.float32)
                    + b2_ref[...])


def _node_mlp(x, sf, w1x, w1s, b1, w2, b2, block_rows=2000):
    n, d = x.shape
    ad = sf.shape[1]
    grid = n // block_rows
    return pl.pallas_call(
        _mlp1_body,
        grid=(grid,),
        in_specs=[
            pl.BlockSpec((block_rows, d), lambda i: (i, 0)),
            pl.BlockSpec((block_rows, ad), lambda i: (i, 0)),
            pl.BlockSpec((d, d), lambda i: (0, 0)),
            pl.BlockSpec((ad, d), lambda i: (0, 0)),
            pl.BlockSpec((1, d), lambda i: (0, 0)),
            pl.BlockSpec((d, d), lambda i: (0, 0)),
            pl.BlockSpec((1, d), lambda i: (0, 0)),
        ],
        out_specs=pl.BlockSpec((block_rows, d), lambda i: (i, 0)),
        out_shape=jax.ShapeDtypeStruct((n, d), jnp.float32),
    )(x, sf, w1x, w1s, b1, w2, b2)


def _bounds_body(seg_ref, out_ref):
    ids = seg_ref[...]
    for j in range(NW + 1):
        t = G if j == NW else j * GPW
        cnt = jnp.sum((ids < t).astype(jnp.int32))
        out_ref[j, :] = jnp.full((16,), 1, jnp.int32) * cnt


def _bounds(seg_pad2d):
    r, ccols = seg_pad2d.shape
    return pl.pallas_call(
        _bounds_body,
        grid=(1,),
        in_specs=[pl.BlockSpec((r, ccols), lambda i: (0, 0))],
        out_specs=pl.BlockSpec((NW + 2, 16), lambda i: (0, 0)),
        out_shape=jax.ShapeDtypeStruct((NW + 2, 16), jnp.int32),
    )(seg_pad2d)


def _make_seg_kernel(n, d):
    max_chunks = n // CH + 1
    mesh = plsc.VectorSubcoreMesh(core_axis_name="c", subcore_axis_name="s")
    nj = d // 16

    @functools.partial(
        pl.kernel,
        out_type=(
            jax.ShapeDtypeStruct((G, d), jnp.float32),
            jax.ShapeDtypeStruct((G, CNT_W), jnp.float32),
        ),
        mesh=mesh,
        scratch_types=[
            pltpu.VMEM((CH, d), jnp.float32),
            pltpu.VMEM((CH,), jnp.int32),
            pltpu.VMEM((NW + 2, 16), jnp.int32),
            pltpu.VMEM((GLAST + 8, d), jnp.float32),
            pltpu.VMEM((GLAST + 8, CNT_W), jnp.float32),
        ],
    )
    def seg_kernel(h_hbm, seg_hbm, bounds_hbm, sums_hbm, cnt_hbm,
                   rows_v, idx_v, bounds_v, acc_v, cnt_v):
        c = lax.axis_index("c")
        s = lax.axis_index("s")
        wid = s * NC + c
        g0 = wid * GPW

        def zero_body(i, carry):
            for j in range(nj):
                acc_v[i, pl.ds(j * 16, 16)] = jnp.zeros((16,), jnp.float32)
            cnt_v[i, :] = jnp.zeros((16,), jnp.float32)
            return carry
        lax.fori_loop(0, GLAST + 8, zero_body, 0)

        pltpu.sync_copy(bounds_hbm, bounds_v)
        rlo = bounds_v[wid, :][0]
        rhi = bounds_v[wid + 1, :][0]
        rbase = (rlo // 8) * 8

        ones16 = jnp.full((16,), 1.0, jnp.float32)

        def chunk_body(i, carry):
            start0 = rbase + i * CH
            start = jnp.minimum(start0, jnp.int32(n - CH))

            @pl.when(start0 < rhi)
            def _process():
                pltpu.sync_copy(h_hbm.at[pl.ds(start, CH)], rows_v)
                pltpu.sync_copy(seg_hbm.at[pl.ds(start, CH)], idx_v)
                lo_valid = jnp.maximum(rlo, start0)

                for k in range(CH // 16):
                    ids16 = idx_v[pl.ds(k * 16, 16)]
                    for lane in range(16):
                        r = k * 16 + lane
                        ridx = start + r
                        valid = jnp.logical_and(ridx >= lo_valid, ridx < rhi)
                        gid = ids16[lane]
                        slot = jnp.where(valid, gid - g0, jnp.int32(GLAST))
                        for j in range(nj):
                            sl = pl.ds(j * 16, 16)
                            acc_v[slot, sl] = acc_v[slot, sl] + rows_v[r, sl]
                        cnt_v[slot, :] = cnt_v[slot, :] + ones16

            return carry

        lax.fori_loop(0, max_chunks, chunk_body, 0)

        pltpu.sync_copy(acc_v.at[pl.ds(0, GPW)], sums_hbm.at[pl.ds(g0, GPW)])
        pltpu.sync_copy(cnt_v.at[pl.ds(0, GPW)], cnt_hbm.at[pl.ds(g0, GPW)])

        @pl.when(wid == NW - 1)
        def _write_tail():
            tb = NW * GPW
            pltpu.sync_copy(acc_v.at[pl.ds(GPW, GLAST - GPW)],
                            sums_hbm.at[pl.ds(tb, GLAST - GPW)])
            pltpu.sync_copy(cnt_v.at[pl.ds(GPW, GLAST - GPW)],
                            cnt_hbm.at[pl.ds(tb, GLAST - GPW)])

    return seg_kernel


def _head_body(sums_ref, cnt_ref, wc1_ref, bc1_ref, wc2_ref, bc2_ref, out_ref):
    count = cnt_ref[:, 0:1]
    pooled = sums_ref[...] / jnp.maximum(count, 1.0)
    g = jax.nn.gelu(jnp.dot(pooled, wc1_ref[...],
                            preferred_element_type=jnp.float32) + bc1_ref[...])
    out_ref[...] = (jnp.dot(g, wc2_ref[...], preferred_element_type=jnp.float32)
                    + bc2_ref[...])


def _head(sums, cnt, wc1, bc1, wc2, bc2, block_rows=2000):
    g, d = sums.shape
    c_out = wc2.shape[1]
    grid = g // block_rows
    return pl.pallas_call(
        _head_body,
        grid=(grid,),
        in_specs=[
            pl.BlockSpec((block_rows, d), lambda i: (i, 0)),
            pl.BlockSpec((block_rows, CNT_W), lambda i: (i, 0)),
            pl.BlockSpec((d, d), lambda i: (0, 0)),
            pl.BlockSpec((1, d), lambda i: (0, 0)),
            pl.BlockSpec((d, c_out), lambda i: (0, 0)),
            pl.BlockSpec((1, c_out), lambda i: (0, 0)),
        ],
        out_specs=pl.BlockSpec((block_rows, c_out), lambda i: (i, 0)),
        out_shape=jax.ShapeDtypeStruct((g, c_out), jnp.float32),
    )(sums, cnt, wc1, bc1, wc2, bc2)


def kernel(x, segment_ids, stage_features, W_af1, b_af1, W_af2, b_af2,
           W_c1, b_c1, W_c2, b_c2):
    n, d = x.shape
    w1x = W_af1[:d]
    w1s = W_af1[d:]
    seg = segment_ids.astype(jnp.int32)
    h = _node_mlp(x, stage_features, w1x, w1s, b_af1.reshape(1, d),
                  W_af2, b_af2.reshape(1, d))
    npad = ((n + 127) // 128) * 128
    seg_pad2d = jnp.concatenate(
        [seg, jnp.full((npad - n,), G, jnp.int32)]).reshape(npad // 128, 128)
    bounds = _bounds(seg_pad2d)
    sums, cnt = _make_seg_kernel(n, d)(h, seg, bounds)
    return _head(sums, cnt, W_c1, b_c1.reshape(1, d),
                 W_c2, b_c2.reshape(1, W_c2.shape[1]))


# speedup vs baseline: 1.6969x; 1.0964x over previous
import functools

import jax
import jax.numpy as jnp
from jax import lax
from jax.experimental import pallas as pl
from jax.experimental.pallas import tpu as pltpu
from jax.experimental.pallas import tpu_sc as plsc

G = 10000
NC = 2
NS = 16
NW = NC * NS
GPW = 312
GLAST = G - (NW - 1) * GPW
CH = 80
CNT_W = 16


def _mlp1_body(x_ref, sf_ref, w1x_ref, w1s_ref, b1_ref, w2_ref, b2_ref, out_ref):
    xb = x_ref[...].astype(jnp.bfloat16)
    sb = sf_ref[...].astype(jnp.bfloat16)
    h = (jnp.dot(xb, w1x_ref[...].astype(jnp.bfloat16),
                 preferred_element_type=jnp.float32)
         + jnp.dot(sb, w1s_ref[...].astype(jnp.bfloat16),
                   preferred_element_type=jnp.float32)
         + b1_ref[...])
    g = jax.nn.gelu(h).astype(jnp.bfloat16)
    out_ref[...] = (jnp.dot(g, w2_ref[...].astype(jnp.bfloat16),
                            preferred_element_type=jnp.float32)
                    + b2_ref[...])


def _node_mlp(x, sf, w1x, w1s, b1, w2, b2, block_rows=2000):
    n, d = x.shape
    ad = sf.shape[1]
    grid = n // block_rows
    return pl.pallas_call(
        _mlp1_body,
        grid=(grid,),
        in_specs=[
            pl.BlockSpec((block_rows, d), lambda i: (i, 0)),
            pl.BlockSpec((block_rows, ad), lambda i: (i, 0)),
            pl.BlockSpec((d, d), lambda i: (0, 0)),
            pl.BlockSpec((ad, d), lambda i: (0, 0)),
            pl.BlockSpec((1, d), lambda i: (0, 0)),
            pl.BlockSpec((d, d), lambda i: (0, 0)),
            pl.BlockSpec((1, d), lambda i: (0, 0)),
        ],
        out_specs=pl.BlockSpec((block_rows, d), lambda i: (i, 0)),
        out_shape=jax.ShapeDtypeStruct((n, d), jnp.float32),
    )(x, sf, w1x, w1s, b1, w2, b2)


def _bounds_body(seg_ref, out_ref):
    ids = seg_ref[...]
    for j in range(NW + 1):
        t = G if j == NW else j * GPW
        cnt = jnp.sum((ids < t).astype(jnp.int32))
        out_ref[j, :] = jnp.full((16,), 1, jnp.int32) * cnt


def _bounds(seg_pad2d):
    r, ccols = seg_pad2d.shape
    return pl.pallas_call(
        _bounds_body,
        grid=(1,),
        in_specs=[pl.BlockSpec((r, ccols), lambda i: (0, 0))],
        out_specs=pl.BlockSpec((NW + 2, 16), lambda i: (0, 0)),
        out_shape=jax.ShapeDtypeStruct((NW + 2, 16), jnp.int32),
    )(seg_pad2d)


def _make_seg_kernel(n, d):
    max_chunks = n // CH + 1
    mesh = plsc.VectorSubcoreMesh(core_axis_name="c", subcore_axis_name="s")
    nj = d // 16

    @functools.partial(
        pl.kernel,
        out_type=(
            jax.ShapeDtypeStruct((G, d), jnp.float32),
            jax.ShapeDtypeStruct((G, CNT_W), jnp.float32),
        ),
        mesh=mesh,
        scratch_types=[
            pltpu.VMEM((CH, d), jnp.float32),
            pltpu.VMEM((CH, d), jnp.float32),
            pltpu.VMEM((CH,), jnp.int32),
            pltpu.VMEM((CH,), jnp.int32),
            pltpu.SemaphoreType.DMA,
            pltpu.SemaphoreType.DMA,
            pltpu.VMEM((NW + 2, 16), jnp.int32),
            pltpu.VMEM((GLAST + 8, d), jnp.float32),
            pltpu.VMEM((GLAST + 8, CNT_W), jnp.float32),
        ],
    )
    def seg_kernel(h_hbm, seg_hbm, bounds_hbm, sums_hbm, cnt_hbm,
                   rows0_v, rows1_v, idx0_v, idx1_v, sem0, sem1, bounds_v,
                   acc_v, cnt_v):
        c = lax.axis_index("c")
        s = lax.axis_index("s")
        wid = s * NC + c
        g0 = wid * GPW

        def zero_body(i, carry):
            for j in range(nj):
                acc_v[i, pl.ds(j * 16, 16)] = jnp.zeros((16,), jnp.float32)
            cnt_v[i, :] = jnp.zeros((16,), jnp.float32)
            return carry
        lax.fori_loop(0, GLAST + 8, zero_body, 0)

        pltpu.sync_copy(bounds_hbm, bounds_v)
        rlo = bounds_v[wid, :][0]
        rhi = bounds_v[wid + 1, :][0]
        rbase = (rlo // 8) * 8

        ones16 = jnp.full((16,), 1.0, jnp.float32)
        bufs = ((rows0_v, idx0_v, sem0), (rows1_v, idx1_v, sem1))
        nlast = jnp.int32(n - CH)

        def chunk_start(ci_val):
            return jnp.minimum(rbase + ci_val * CH, nlast)

        def start_dma(ci_val, rows_v, idx_v, sem):
            st = chunk_start(ci_val)
            pltpu.async_copy(h_hbm.at[pl.ds(st, CH)], rows_v, sem)
            pltpu.async_copy(seg_hbm.at[pl.ds(st, CH)], idx_v, sem)

        def wait_dma(rows_v, idx_v, sem):
            pltpu.make_async_copy(h_hbm.at[pl.ds(0, CH)], rows_v, sem).wait()
            pltpu.make_async_copy(seg_hbm.at[pl.ds(0, CH)], idx_v, sem).wait()

        for b in range(2):

            @pl.when(rbase + b * CH < rhi)
            def _prime():
                start_dma(jnp.int32(b), *bufs[b])

        def chunk_pair(i2, carry):
            for b in range(2):
                ci = i2 * 2 + b
                start0 = rbase + ci * CH
                rows_v, idx_v, sem = bufs[b]

                @pl.when(start0 < rhi)
                def _process():
                    wait_dma(rows_v, idx_v, sem)
                    start = chunk_start(ci)
                    lo_valid = jnp.maximum(rlo, start0)

                    for k in range(CH // 16):
                        ids16 = idx_v[pl.ds(k * 16, 16)]
                        for lane in range(16):
                            r = k * 16 + lane
                            ridx = start + r
                            valid = jnp.logical_and(
                                ridx >= lo_valid, ridx < rhi)
                            gid = ids16[lane]
                            slot = jnp.where(valid, gid - g0,
                                             jnp.int32(GLAST))
                            for j in range(nj):
                                sl = pl.ds(j * 16, 16)
                                acc_v[slot, sl] = (acc_v[slot, sl]
                                                   + rows_v[r, sl])
                            cnt_v[slot, :] = cnt_v[slot, :] + ones16

                    @pl.when(start0 + 2 * CH < rhi)
                    def _prefetch():
                        start_dma(ci + 2, rows_v, idx_v, sem)

            return carry

        lax.fori_loop(0, max_chunks // 2 + 1, chunk_pair, 0)

        pltpu.sync_copy(acc_v.at[pl.ds(0, GPW)], sums_hbm.at[pl.ds(g0, GPW)])
        pltpu.sync_copy(cnt_v.at[pl.ds(0, GPW)], cnt_hbm.at[pl.ds(g0, GPW)])

        @pl.when(wid == NW - 1)
        def _write_tail():
            tb = NW * GPW
            pltpu.sync_copy(acc_v.at[pl.ds(GPW, GLAST - GPW)],
                            sums_hbm.at[pl.ds(tb, GLAST - GPW)])
            pltpu.sync_copy(cnt_v.at[pl.ds(GPW, GLAST - GPW)],
                            cnt_hbm.at[pl.ds(tb, GLAST - GPW)])

    return seg_kernel


def _head_body(sums_ref, cnt_ref, wc1_ref, bc1_ref, wc2_ref, bc2_ref, out_ref):
    count = cnt_ref[:, 0:1]
    pooled = sums_ref[...] / jnp.maximum(count, 1.0)
    g = jax.nn.gelu(jnp.dot(pooled, wc1_ref[...],
                            preferred_element_type=jnp.float32) + bc1_ref[...])
    out_ref[...] = (jnp.dot(g, wc2_ref[...], preferred_element_type=jnp.float32)
                    + bc2_ref[...])


def _head(sums, cnt, wc1, bc1, wc2, bc2, block_rows=2000):
    g, d = sums.shape
    c_out = wc2.shape[1]
    grid = g // block_rows
    return pl.pallas_call(
        _head_body,
        grid=(grid,),
        in_specs=[
            pl.BlockSpec((block_rows, d), lambda i: (i, 0)),
            pl.BlockSpec((block_rows, CNT_W), lambda i: (i, 0)),
            pl.BlockSpec((d, d), lambda i: (0, 0)),
            pl.BlockSpec((1, d), lambda i: (0, 0)),
            pl.BlockSpec((d, c_out), lambda i: (0, 0)),
            pl.BlockSpec((1, c_out), lambda i: (0, 0)),
        ],
        out_specs=pl.BlockSpec((block_rows, c_out), lambda i: (i, 0)),
        out_shape=jax.ShapeDtypeStruct((g, c_out), jnp.float32),
    )(sums, cnt, wc1, bc1, wc2, bc2)


def kernel(x, segment_ids, stage_features, W_af1, b_af1, W_af2, b_af2,
           W_c1, b_c1, W_c2, b_c2):
    n, d = x.shape
    w1x = W_af1[:d]
    w1s = W_af1[d:]
    seg = segment_ids.astype(jnp.int32)
    h = _node_mlp(x, stage_features, w1x, w1s, b_af1.reshape(1, d),
                  W_af2, b_af2.reshape(1, d))
    npad = ((n + 127) // 128) * 128
    seg_pad2d = jnp.concatenate(
        [seg, jnp.full((npad - n,), G, jnp.int32)]).reshape(npad // 128, 128)
    bounds = _bounds(seg_pad2d)
    sums, cnt = _make_seg_kernel(n, d)(h, seg, bounds)
    return _head(sums, cnt, W_c1, b_c1.reshape(1, d),
                 W_c2, b_c2.reshape(1, W_c2.shape[1]))


# speedup vs baseline: 1.7319x; 1.0207x over previous
import functools

import jax
import jax.numpy as jnp
from jax import lax
from jax.experimental import pallas as pl
from jax.experimental.pallas import tpu as pltpu
from jax.experimental.pallas import tpu_sc as plsc

G = 10000
NC = 2
NS = 16
NW = NC * NS
GPW = 312
GLAST = G - (NW - 1) * GPW
CH = 80
CNT_W = 16


def _mlp1_body(x_ref, sf_ref, w1x_ref, w1s_ref, b1_ref, w2_ref, b2_ref, out_ref):
    h = (jnp.dot(x_ref[...], w1x_ref[...], preferred_element_type=jnp.float32)
         + jnp.dot(sf_ref[...], w1s_ref[...], preferred_element_type=jnp.float32)
         + b1_ref[...])
    g = jax.nn.gelu(h)
    out_ref[...] = (jnp.dot(g, w2_ref[...], preferred_element_type=jnp.float32)
                    + b2_ref[...])


def _node_mlp(x, sf, w1x, w1s, b1, w2, b2, block_rows=2000):
    n, d = x.shape
    ad = sf.shape[1]
    grid = n // block_rows
    return pl.pallas_call(
        _mlp1_body,
        grid=(grid,),
        in_specs=[
            pl.BlockSpec((block_rows, d), lambda i: (i, 0)),
            pl.BlockSpec((block_rows, ad), lambda i: (i, 0)),
            pl.BlockSpec((d, d), lambda i: (0, 0)),
            pl.BlockSpec((ad, d), lambda i: (0, 0)),
            pl.BlockSpec((1, d), lambda i: (0, 0)),
            pl.BlockSpec((d, d), lambda i: (0, 0)),
            pl.BlockSpec((1, d), lambda i: (0, 0)),
        ],
        out_specs=pl.BlockSpec((block_rows, d), lambda i: (i, 0)),
        out_shape=jax.ShapeDtypeStruct((n, d), jnp.float32),
    )(x, sf, w1x, w1s, b1, w2, b2)


def _bounds_body(seg_ref, out_ref):
    ids = seg_ref[...]
    for j in range(NW + 1):
        t = G if j == NW else j * GPW
        cnt = jnp.sum((ids < t).astype(jnp.int32))
        out_ref[j, :] = jnp.full((16,), 1, jnp.int32) * cnt


def _bounds(seg_pad2d):
    r, ccols = seg_pad2d.shape
    return pl.pallas_call(
        _bounds_body,
        grid=(1,),
        in_specs=[pl.BlockSpec((r, ccols), lambda i: (0, 0))],
        out_specs=pl.BlockSpec((NW + 2, 16), lambda i: (0, 0)),
        out_shape=jax.ShapeDtypeStruct((NW + 2, 16), jnp.int32),
    )(seg_pad2d)


def _make_seg_kernel(n, d):
    max_chunks = n // CH + 1
    mesh = plsc.VectorSubcoreMesh(core_axis_name="c", subcore_axis_name="s")
    nj = d // 16

    @functools.partial(
        pl.kernel,
        out_type=(
            jax.ShapeDtypeStruct((G, d), jnp.float32),
            jax.ShapeDtypeStruct((G, CNT_W), jnp.float32),
        ),
        mesh=mesh,
        scratch_types=[
            pltpu.VMEM((CH, d), jnp.float32),
            pltpu.VMEM((CH, d), jnp.float32),
            pltpu.VMEM((CH,), jnp.int32),
            pltpu.VMEM((CH,), jnp.int32),
            pltpu.SemaphoreType.DMA,
            pltpu.SemaphoreType.DMA,
            pltpu.VMEM((NW + 2, 16), jnp.int32),
            pltpu.VMEM((GLAST + 8, d), jnp.float32),
            pltpu.VMEM((GLAST + 8, CNT_W), jnp.float32),
        ],
    )
    def seg_kernel(h_hbm, seg_hbm, bounds_hbm, sums_hbm, cnt_hbm,
                   rows0_v, rows1_v, idx0_v, idx1_v, sem0, sem1, bounds_v,
                   acc_v, cnt_v):
        c = lax.axis_index("c")
        s = lax.axis_index("s")
        wid = s * NC + c
        g0 = wid * GPW

        def zero_body(i, carry):
            for j in range(nj):
                acc_v[i, pl.ds(j * 16, 16)] = jnp.zeros((16,), jnp.float32)
            cnt_v[i, :] = jnp.zeros((16,), jnp.float32)
            return carry
        lax.fori_loop(0, GLAST + 8, zero_body, 0)

        pltpu.sync_copy(bounds_hbm, bounds_v)
        rlo = bounds_v[wid, :][0]
        rhi = bounds_v[wid + 1, :][0]
        rbase = (rlo // 8) * 8

        ones16 = jnp.full((16,), 1.0, jnp.float32)
        bufs = ((rows0_v, idx0_v, sem0), (rows1_v, idx1_v, sem1))
        nlast = jnp.int32(n - CH)

        def chunk_start(ci_val):
            return jnp.minimum(rbase + ci_val * CH, nlast)

        def start_dma(ci_val, rows_v, idx_v, sem):
            st = chunk_start(ci_val)
            pltpu.async_copy(h_hbm.at[pl.ds(st, CH)], rows_v, sem)
            pltpu.async_copy(seg_hbm.at[pl.ds(st, CH)], idx_v, sem)

        def wait_dma(rows_v, idx_v, sem):
            pltpu.make_async_copy(h_hbm.at[pl.ds(0, CH)], rows_v, sem).wait()
            pltpu.make_async_copy(seg_hbm.at[pl.ds(0, CH)], idx_v, sem).wait()

        for b in range(2):

            @pl.when(rbase + b * CH < rhi)
            def _prime():
                start_dma(jnp.int32(b), *bufs[b])

        def chunk_pair(i2, carry):
            for b in range(2):
                ci = i2 * 2 + b
                start0 = rbase + ci * CH
                rows_v, idx_v, sem = bufs[b]

                @pl.when(start0 < rhi)
                def _process():
                    wait_dma(rows_v, idx_v, sem)
                    start = chunk_start(ci)
                    lo_valid = jnp.maximum(rlo, start0)

                    for k in range(CH // 16):
                        ids16 = idx_v[pl.ds(k * 16, 16)]
                        for lane in range(16):
                            r = k * 16 + lane
                            ridx = start + r
                            valid = jnp.logical_and(
                                ridx >= lo_valid, ridx < rhi)
                            gid = ids16[lane]
                            slot = jnp.where(valid, gid - g0,
                                             jnp.int32(GLAST))
                            for j in range(nj):
                                sl = pl.ds(j * 16, 16)
                                acc_v[slot, sl] = (acc_v[slot, sl]
                                                   + rows_v[r, sl])
                            cnt_v[slot, :] = cnt_v[slot, :] + ones16

                    @pl.when(start0 + 2 * CH < rhi)
                    def _prefetch():
                        start_dma(ci + 2, rows_v, idx_v, sem)

            return carry

        lax.fori_loop(0, max_chunks // 2 + 1, chunk_pair, 0)

        pltpu.sync_copy(acc_v.at[pl.ds(0, GPW)], sums_hbm.at[pl.ds(g0, GPW)])
        pltpu.sync_copy(cnt_v.at[pl.ds(0, GPW)], cnt_hbm.at[pl.ds(g0, GPW)])

        @pl.when(wid == NW - 1)
        def _write_tail():
            tb = NW * GPW
            pltpu.sync_copy(acc_v.at[pl.ds(GPW, GLAST - GPW)],
                            sums_hbm.at[pl.ds(tb, GLAST - GPW)])
            pltpu.sync_copy(cnt_v.at[pl.ds(GPW, GLAST - GPW)],
                            cnt_hbm.at[pl.ds(tb, GLAST - GPW)])

    return seg_kernel


def _head_body(sums_ref, cnt_ref, wc1_ref, bc1_ref, wc2_ref, bc2_ref, out_ref):
    count = cnt_ref[:, 0:1]
    pooled = sums_ref[...] / jnp.maximum(count, 1.0)
    g = jax.nn.gelu(jnp.dot(pooled, wc1_ref[...],
                            preferred_element_type=jnp.float32) + bc1_ref[...])
    out_ref[...] = (jnp.dot(g, wc2_ref[...], preferred_element_type=jnp.float32)
                    + bc2_ref[...])


def _head(sums, cnt, wc1, bc1, wc2, bc2, block_rows=2000):
    g, d = sums.shape
    c_out = wc2.shape[1]
    grid = g // block_rows
    return pl.pallas_call(
        _head_body,
        grid=(grid,),
        in_specs=[
            pl.BlockSpec((block_rows, d), lambda i: (i, 0)),
            pl.BlockSpec((block_rows, CNT_W), lambda i: (i, 0)),
            pl.BlockSpec((d, d), lambda i: (0, 0)),
            pl.BlockSpec((1, d), lambda i: (0, 0)),
            pl.BlockSpec((d, c_out), lambda i: (0, 0)),
            pl.BlockSpec((1, c_out), lambda i: (0, 0)),
        ],
        out_specs=pl.BlockSpec((block_rows, c_out), lambda i: (i, 0)),
        out_shape=jax.ShapeDtypeStruct((g, c_out), jnp.float32),
    )(sums, cnt, wc1, bc1, wc2, bc2)


def kernel(x, segment_ids, stage_features, W_af1, b_af1, W_af2, b_af2,
           W_c1, b_c1, W_c2, b_c2):
    n, d = x.shape
    w1x = W_af1[:d]
    w1s = W_af1[d:]
    seg = segment_ids.astype(jnp.int32)
    h = _node_mlp(x, stage_features, w1x, w1s, b_af1.reshape(1, d),
                  W_af2, b_af2.reshape(1, d))
    npad = ((n + 127) // 128) * 128
    seg_pad2d = jnp.concatenate(
        [seg, jnp.full((npad - n,), G, jnp.int32)]).reshape(npad // 128, 128)
    bounds = _bounds(seg_pad2d)
    sums, cnt = _make_seg_kernel(n, d)(h, seg, bounds)
    return _head(sums, cnt, W_c1, b_c1.reshape(1, d),
                 W_c2, b_c2.reshape(1, W_c2.shape[1]))
